# baseline (speedup 1.0000x reference)
import functools

import jax
import jax.numpy as jnp
from jax import lax
from jax.experimental import pallas as pl
from jax.experimental.pallas import tpu as pltpu
from jax.experimental.pallas import tpu_sc as plsc

_B, _T, _D = 16384, 200, 4
_NW = 32
_BT = _B // 128
_BT_W = _BT // _NW
_TSEG = 50
_NSEG = _T // _TSEG
_NCHUNK = _BT_W * _NSEG


def _sc_embed(idx, table):
    mesh = plsc.VectorSubcoreMesh(core_axis_name="c", subcore_axis_name="s")

    @functools.partial(
        pl.kernel,
        mesh=mesh,
        out_type=jax.ShapeDtypeStruct((_T, _BT, _D, 128), jnp.float32),
        compiler_params=pltpu.CompilerParams(
            needs_layout_passes=False,
            use_tc_tiling_on_sc=False,
        ),
        scratch_types=[
            pltpu.VMEM((8, 4), jnp.float32),
            pltpu.VMEM((128, _T), jnp.int32),
            pltpu.VMEM((2 * _TSEG, 1, _D, 128), jnp.float32),
            pltpu.SemaphoreType.DMA,
            pltpu.SemaphoreType.DMA,
        ],
    )
    def k(idx_hbm, tab_hbm, out_hbm, tab_v, idx_v, out_v, sem0, sem1):
        wid = lax.axis_index("s") * 2 + lax.axis_index("c")
        pltpu.sync_copy(tab_hbm, tab_v)
        lane = lax.iota(jnp.int32, 16)

        def out_slices(c):
            bt = wid * _BT_W + lax.shift_right_logical(c, 2)
            t0 = lax.bitwise_and(c, 3) * _TSEG
            src = out_v.at[pl.ds(lax.bitwise_and(c, 1) * _TSEG, _TSEG)]
            dst = out_hbm.at[pl.ds(t0, _TSEG), pl.ds(bt, 1), :, :]
            return src, dst

        def chunk_body(c, _):
            bt = wid * _BT_W + lax.shift_right_logical(c, 2)
            seg = lax.bitwise_and(c, 3)
            par = lax.bitwise_and(c, 1)
            t0 = seg * _TSEG
            v0 = par * _TSEG

            @pl.when(seg == 0)
            def _():
                pltpu.sync_copy(idx_hbm.at[pl.ds(bt * 128, 128), :], idx_v)

            @pl.when(jnp.logical_and(c >= 2, par == 0))
            def _():
                src, dst = out_slices(c - 2)
                pltpu.make_async_copy(src, dst, sem0).wait()

            @pl.when(jnp.logical_and(c >= 2, par == 1))
            def _():
                src, dst = out_slices(c - 2)
                pltpu.make_async_copy(src, dst, sem1).wait()

            @plsc.parallel_loop(0, _TSEG, unroll=4)
            def t_body(tt):
                tcol = jnp.broadcast_to(t0 + tt, (16,))
                for g in range(8):
                    rows = jnp.broadcast_to(g * 16, (16,)) + lane
                    idxg = plsc.load_gather(idx_v, [rows, tcol])
                    for kk in range(4):
                        vals = plsc.load_gather(
                            tab_v, [idxg, jnp.broadcast_to(kk, (16,))]
                        )
                        out_v[v0 + tt, 0, kk, pl.ds(g * 16, 16)] = vals

            src, dst = out_slices(c)

            @pl.when(par == 0)
            def _():
                pltpu.async_copy(src, dst, sem0)

            @pl.when(par == 1)
            def _():
                pltpu.async_copy(src, dst, sem1)

            return 0

        lax.fori_loop(0, _NCHUNK, chunk_body, 0)

        src, dst = out_slices(_NCHUNK - 2)
        pltpu.make_async_copy(src, dst, sem0).wait()
        src, dst = out_slices(_NCHUNK - 1)
        pltpu.make_async_copy(src, dst, sem1).wait()

    return k(idx, table)


def kernel(enc_input_ids, dec_input_ids, dec_embed):
    del enc_input_ids
    out_phys = _sc_embed(dec_input_ids, dec_embed)
    return out_phys.transpose(1, 3, 0, 2).reshape(_B, _T, _D)


# speedup vs baseline: 1.9721x; 1.9721x over previous
import functools

import jax
import jax.numpy as jnp
from jax import lax
from jax.experimental import pallas as pl
from jax.experimental.pallas import tpu as pltpu
from jax.experimental.pallas import tpu_sc as plsc

_B, _T, _D = 16384, 200, 4
_NW = 32
_BT = _B // 128
_BT_W = _BT // _NW
_TSEG = 50
_NSEG = _T // _TSEG
_NCHUNK = _BT_W * _NSEG


def _sc_embed(idx, table):
    mesh = plsc.VectorSubcoreMesh(core_axis_name="c", subcore_axis_name="s")

    @functools.partial(
        pl.kernel,
        mesh=mesh,
        out_type=jax.ShapeDtypeStruct((_T, _BT, _D, 128), jnp.float32),
        compiler_params=pltpu.CompilerParams(
            needs_layout_passes=False,
            use_tc_tiling_on_sc=False,
        ),
        scratch_types=[
            pltpu.VMEM((8, 4), jnp.float32),
            pltpu.VMEM((128, _T), jnp.int32),
            pltpu.VMEM((2 * _TSEG, 1, _D, 128), jnp.float32),
            pltpu.SemaphoreType.DMA,
            pltpu.SemaphoreType.DMA,
        ],
    )
    def k(idx_hbm, tab_hbm, out_hbm, tab_v, idx_v, out_v, sem0, sem1):
        wid = lax.axis_index("s") * 2 + lax.axis_index("c")
        pltpu.sync_copy(tab_hbm, tab_v)
        lane = lax.iota(jnp.int32, 16)

        def out_slices(c):
            bt = wid * _BT_W + lax.shift_right_logical(c, 2)
            t0 = lax.bitwise_and(c, 3) * _TSEG
            src = out_v.at[pl.ds(lax.bitwise_and(c, 1) * _TSEG, _TSEG)]
            dst = out_hbm.at[pl.ds(t0, _TSEG), pl.ds(bt, 1), :, :]
            return src, dst

        def chunk_body(c, _):
            bt = wid * _BT_W + lax.shift_right_logical(c, 2)
            seg = lax.bitwise_and(c, 3)
            par = lax.bitwise_and(c, 1)
            t0 = seg * _TSEG
            v0 = par * _TSEG

            @pl.when(seg == 0)
            def _():
                pltpu.sync_copy(idx_hbm.at[pl.ds(bt * 128, 128), :], idx_v)

            @pl.when(jnp.logical_and(c >= 2, par == 0))
            def _():
                src, dst = out_slices(c - 2)
                pltpu.make_async_copy(src, dst, sem0).wait()

            @pl.when(jnp.logical_and(c >= 2, par == 1))
            def _():
                src, dst = out_slices(c - 2)
                pltpu.make_async_copy(src, dst, sem1).wait()

            @plsc.parallel_loop(0, _TSEG, unroll=2)
            def t_body(tt):
                tcol = jnp.broadcast_to(t0 + tt, (16,))
                for g in range(8):
                    rows = jnp.broadcast_to(g * 16, (16,)) + lane
                    idxg = plsc.load_gather(idx_v, [rows, tcol])
                    for kk in range(4):
                        vals = plsc.load_gather(
                            tab_v, [idxg, jnp.broadcast_to(kk, (16,))]
                        )
                        out_v[v0 + tt, 0, kk, pl.ds(g * 16, 16)] = vals

            src, dst = out_slices(c)

            @pl.when(par == 0)
            def _():
                pltpu.async_copy(src, dst, sem0)

            @pl.when(par == 1)
            def _():
                pltpu.async_copy(src, dst, sem1)

            return 0

        lax.fori_loop(0, _NCHUNK, chunk_body, 0)

        src, dst = out_slices(_NCHUNK - 2)
        pltpu.make_async_copy(src, dst, sem0).wait()
        src, dst = out_slices(_NCHUNK - 1)
        pltpu.make_async_copy(src, dst, sem1).wait()

    return k(idx, table)


def kernel(enc_input_ids, dec_input_ids, dec_embed):
    del enc_input_ids
    out_phys = _sc_embed(dec_input_ids, dec_embed)
    return out_phys.transpose(1, 3, 0, 2).reshape(_B, _T, _D)
